# baseline (speedup 1.0000x reference)
import jax
import jax.numpy as jnp
from jax import lax
from jax.experimental import pallas as pl
from jax.experimental.pallas import tpu as pltpu
from jax.experimental.pallas import tpu_sc as plsc

NC, NS, L = 1, 16, 16
NW = NC * NS
B, H, W, D = 8, 256, 256, 128
PER_SAMPLE = H * W
TOTAL = B * PER_SAMPLE
CHUNK = TOTAL // NW
ROWS = CHUNK // L


def _body(r3d_hbm, in2d_hbm, idx_hbm, out_hbm,
          idx16_v, idxbuf_v, gbuf_v, ybuf_v, acc_v, gsem, ysem):
    c = lax.axis_index("c")
    s = lax.axis_index("s")
    wid = s * NC + c

    ycopy = pltpu.async_copy(in2d_hbm.at[pl.ds(wid * CHUNK, CHUNK)], ybuf_v, ysem)

    pltpu.sync_copy(idx_hbm.at[wid], idx16_v)
    lanes = lax.broadcasted_iota(jnp.int32, (L,), 0)
    base = idx16_v[...] + lanes * D + wid * (CHUNK * D)

    def build(t, carry):
        idxbuf_v[pl.ds(t * L, L)] = base + t * (L * D)
        return carry
    lax.fori_loop(0, ROWS, build, 0, unroll=8)

    gcopy = pltpu.async_copy(r3d_hbm.at[idxbuf_v], gbuf_v, gsem)
    gcopy.wait()
    ycopy.wait()

    def red(t, acc):
        d = gbuf_v[pl.ds(t * L, L)] - ybuf_v[pl.ds(t * L, L)]
        return acc + d * d
    acc = lax.fori_loop(0, ROWS, red, jnp.zeros((L,), jnp.float32), unroll=8)
    acc_v[...] = acc
    pltpu.sync_copy(acc_v, out_hbm.at[wid])


def kernel(reconstructed_3d, input_2d, slice_idx):
    r3d_flat = reconstructed_3d.reshape(-1)
    in2d = input_2d.reshape(-1)
    idx = jnp.broadcast_to(
        slice_idx.astype(jnp.int32)[:, None, None], (B, NW // B, L)
    ).reshape(NW, L)
    mesh = plsc.VectorSubcoreMesh(core_axis_name="c", subcore_axis_name="s", num_cores=1)
    partials = pl.kernel(
        _body,
        out_type=jax.ShapeDtypeStruct((NW, L), jnp.float32),
        mesh=mesh,
        scratch_types=[
            pltpu.VMEM((L,), jnp.int32),
            pltpu.VMEM((CHUNK,), jnp.int32),
            pltpu.VMEM((CHUNK,), jnp.float32),
            pltpu.VMEM((CHUNK,), jnp.float32),
            pltpu.VMEM((L,), jnp.float32),
            pltpu.SemaphoreType.DMA,
            pltpu.SemaphoreType.DMA,
        ],
    )(r3d_flat, in2d, idx)
    return jnp.sum(partials) / TOTAL


# speedup vs baseline: 1.5309x; 1.5309x over previous
import jax
import jax.numpy as jnp
from jax import lax
from jax.experimental import pallas as pl
from jax.experimental.pallas import tpu as pltpu
from jax.experimental.pallas import tpu_sc as plsc

NC, NS, L = 2, 16, 16
NW = NC * NS
B, H, W, D = 8, 256, 256, 128
PER_SAMPLE = H * W
TOTAL = B * PER_SAMPLE
CHUNK = TOTAL // NW
YROWS = CHUNK // W
NCH = 4
CSZ = CHUNK // NCH
CROWS = CSZ // L


def _body(r3d_hbm, in2d_hbm, idx_hbm, out_hbm,
          idx16_v, idxbuf_v, gbuf_v, ybuf_v, acc_v, ysem, *gsems):
    c = lax.axis_index("c")
    s = lax.axis_index("s")
    wid = s * NC + c
    i = wid // (NW // B)

    ycopy = pltpu.async_copy(
        in2d_hbm.at[pl.ds(wid * YROWS, YROWS)], ybuf_v, ysem)

    pltpu.sync_copy(idx_hbm, idx16_v.at[pl.ds(0, 8)])
    v = idx16_v[...]
    idx_s = jnp.int32(0)
    for j in range(B):
        idx_s = jnp.where(i == j, v[j], idx_s)

    lanes = lax.broadcasted_iota(jnp.int32, (L,), 0)
    base = idx_s + lanes * D + wid * (CHUNK * D)

    copies = []
    for k in range(NCH):
        def build(t, carry, off=k * CROWS):
            idxbuf_v[pl.ds((off + t) * L, L)] = base + (off + t) * (L * D)
            return carry
        lax.fori_loop(0, CROWS, build, 0, unroll=8)
        copies.append(pltpu.async_copy(
            r3d_hbm.at[idxbuf_v.at[pl.ds(k * CSZ, CSZ)]],
            gbuf_v.at[pl.ds(k * CSZ, CSZ)],
            gsems[k]))
    ycopy.wait()

    acc = jnp.zeros((L,), jnp.float32)
    for k in range(NCH):
        copies[k].wait()

        def red(t, a, off=k * CSZ):
            e = off + t * L
            d = (gbuf_v[pl.ds(e, L)]
                 - ybuf_v[e // W, pl.ds(e % W, L)])
            return a + d * d
        acc = lax.fori_loop(0, CROWS, red, acc, unroll=8)
    acc_v[...] = acc
    pltpu.sync_copy(acc_v, out_hbm.at[wid])


def kernel(reconstructed_3d, input_2d, slice_idx):
    r3d_flat = reconstructed_3d.reshape(-1)
    in2d = input_2d.reshape(B * H, W)
    idx = slice_idx.astype(jnp.int32)
    mesh = plsc.VectorSubcoreMesh(core_axis_name="c", subcore_axis_name="s")
    partials = pl.kernel(
        _body,
        out_type=jax.ShapeDtypeStruct((NW, L), jnp.float32),
        mesh=mesh,
        scratch_types=[
            pltpu.VMEM((L,), jnp.int32),
            pltpu.VMEM((CHUNK,), jnp.int32),
            pltpu.VMEM((CHUNK,), jnp.float32),
            pltpu.VMEM((YROWS, W), jnp.float32),
            pltpu.VMEM((L,), jnp.float32),
            pltpu.SemaphoreType.DMA,
        ] + [pltpu.SemaphoreType.DMA] * NCH,
    )(r3d_flat, in2d, idx)
    return jnp.sum(partials) / TOTAL
